# baseline (device time: 33634 ns/iter reference)
import jax
import jax.numpy as jnp
from jax import lax
from jax.experimental import pallas as pl
from jax.experimental.pallas import tpu as pltpu

N_DEV = 4
B, SQ, SKV, D = 2, 128, 128, 512
H = 8
DH = 64


def kernel(x, Wq, Wo, K_ext, V_ext):
    xb = x.reshape(B * SQ, D).astype(jnp.bfloat16)
    wq = Wq.reshape(D, H, DH).transpose(1, 0, 2).astype(jnp.bfloat16)
    wo = Wo.astype(jnp.bfloat16)
    kk = K_ext.transpose(0, 2, 1, 3).reshape(B * H, SKV, DH).astype(jnp.bfloat16)
    vv = V_ext.transpose(0, 2, 1, 3).reshape(B * H, SKV, DH).astype(jnp.bfloat16)

    def body(x_ref, wq_ref, wo_ref, k_ref, v_ref, out_ref,
             comm_ref, send_sems, recv_sems):
        my = lax.axis_index("i")
        left = lax.rem(my + N_DEV - 1, N_DEV)
        right = lax.rem(my + 1, N_DEV)

        barrier = pltpu.get_barrier_semaphore()
        for nbr in (left, right):
            pl.semaphore_signal(barrier, inc=1, device_id=(nbr,),
                                device_id_type=pl.DeviceIdType.MESH)
        pl.semaphore_wait(barrier, 2)

        xv = x_ref[:]
        partial = jnp.zeros((B * SQ, D), jnp.float32)
        for h in range(H):
            qh = jnp.dot(xv, wq_ref[h], preferred_element_type=jnp.float32)
            obs = []
            for b in range(B):
                qbh = (lax.slice(qh, (b * SQ, 0), ((b + 1) * SQ, DH))
                       * 0.125).astype(jnp.bfloat16)
                s = lax.dot_general(qbh, k_ref[b * H + h],
                                    (((1,), (1,)), ((), ())),
                                    preferred_element_type=jnp.float32)
                m = jnp.max(s, axis=1, keepdims=True)
                p = jnp.exp(s - m)
                l = jnp.sum(p, axis=1, keepdims=True)
                o = jnp.dot(p.astype(jnp.bfloat16), v_ref[b * H + h],
                            preferred_element_type=jnp.float32) / l
                obs.append(o)
            oh = jnp.concatenate(obs, axis=0).astype(jnp.bfloat16)
            partial = partial + jnp.dot(oh, wo_ref[pl.ds(h * DH, DH), :],
                                        preferred_element_type=jnp.float32)

        comm_ref[0] = partial
        acc = partial
        for hop in range(N_DEV - 1):
            rdma = pltpu.make_async_remote_copy(
                src_ref=comm_ref.at[hop],
                dst_ref=comm_ref.at[hop + 1],
                send_sem=send_sems.at[hop],
                recv_sem=recv_sems.at[hop],
                device_id=(right,),
                device_id_type=pl.DeviceIdType.MESH,
            )
            rdma.start()
            rdma.wait()
            acc = acc + comm_ref[hop + 1]
        out_ref[:] = acc.reshape(B, SQ, D)

    return pl.pallas_call(
        body,
        out_shape=jax.ShapeDtypeStruct((B, SQ, D), jnp.float32),
        in_specs=[pl.BlockSpec(memory_space=pltpu.VMEM)] * 5,
        out_specs=pl.BlockSpec(memory_space=pltpu.VMEM),
        scratch_shapes=[
            pltpu.VMEM((N_DEV, B * SQ, D), jnp.float32),
            pltpu.SemaphoreType.DMA((N_DEV - 1,)),
            pltpu.SemaphoreType.DMA((N_DEV - 1,)),
        ],
        compiler_params=pltpu.CompilerParams(collective_id=0),
    )(xb, wq, wo, kk, vv)


# device time: 21577 ns/iter; 1.5588x vs baseline; 1.5588x over previous
import jax
import jax.numpy as jnp
from jax import lax
from jax.experimental import pallas as pl
from jax.experimental.pallas import tpu as pltpu

N_DEV = 4
B, SQ, SKV, D = 2, 128, 128, 512
H = 8
DH = 64


def kernel(x, Wq, Wo, K_ext, V_ext):
    xb = x.reshape(B * SQ, D).astype(jnp.bfloat16)
    wq = Wq.reshape(D, H, DH).transpose(1, 0, 2).astype(jnp.bfloat16)
    wo = Wo.astype(jnp.bfloat16)
    kk = K_ext.transpose(0, 2, 1, 3).reshape(B * H, SKV, DH).astype(jnp.bfloat16)
    vv = V_ext.transpose(0, 2, 1, 3).reshape(B * H, SKV, DH).astype(jnp.bfloat16)

    def body(x_ref, wq_ref, wo_ref, k_ref, v_ref, out_ref,
             send_ref, recv_ref, send_sems, recv_sems):
        my = lax.axis_index("i")
        partners = (jnp.bitwise_xor(my, 1), 3 - my)

        barrier = pltpu.get_barrier_semaphore()
        for nbr in partners:
            pl.semaphore_signal(barrier, inc=1, device_id=(nbr,),
                                device_id_type=pl.DeviceIdType.MESH)
        pl.semaphore_wait(barrier, 2)

        xv = x_ref[:]
        partial = jnp.zeros((B * SQ, D), jnp.float32)
        for h in range(H):
            qh = jnp.dot(xv, wq_ref[h], preferred_element_type=jnp.float32)
            obs = []
            for b in range(B):
                qbh = (lax.slice(qh, (b * SQ, 0), ((b + 1) * SQ, DH))
                       * 0.125).astype(jnp.bfloat16)
                s = lax.dot_general(qbh, k_ref[b * H + h],
                                    (((1,), (1,)), ((), ())),
                                    preferred_element_type=jnp.float32)
                m = jnp.max(s, axis=1, keepdims=True)
                p = jnp.exp(s - m)
                l = jnp.sum(p, axis=1, keepdims=True)
                o = jnp.dot(p.astype(jnp.bfloat16), v_ref[b * H + h],
                            preferred_element_type=jnp.float32) / l
                obs.append(o)
            oh = jnp.concatenate(obs, axis=0).astype(jnp.bfloat16)
            partial = partial + jnp.dot(oh, wo_ref[pl.ds(h * DH, DH), :],
                                        preferred_element_type=jnp.float32)

        acc = partial
        for r in range(2):
            send_ref[r] = acc.astype(jnp.bfloat16)
            rdma = pltpu.make_async_remote_copy(
                src_ref=send_ref.at[r],
                dst_ref=recv_ref.at[r],
                send_sem=send_sems.at[r],
                recv_sem=recv_sems.at[r],
                device_id=(partners[r],),
                device_id_type=pl.DeviceIdType.MESH,
            )
            rdma.start()
            rdma.wait()
            acc = acc + recv_ref[r].astype(jnp.float32)
        out_ref[:] = acc.reshape(B, SQ, D)

    return pl.pallas_call(
        body,
        out_shape=jax.ShapeDtypeStruct((B, SQ, D), jnp.float32),
        in_specs=[pl.BlockSpec(memory_space=pltpu.VMEM)] * 5,
        out_specs=pl.BlockSpec(memory_space=pltpu.VMEM),
        scratch_shapes=[
            pltpu.VMEM((2, B * SQ, D), jnp.bfloat16),
            pltpu.VMEM((2, B * SQ, D), jnp.bfloat16),
            pltpu.SemaphoreType.DMA((2,)),
            pltpu.SemaphoreType.DMA((2,)),
        ],
        compiler_params=pltpu.CompilerParams(collective_id=0),
    )(xb, wq, wo, kk, vv)
